# baseline (speedup 1.0000x reference)
import functools

import jax
import jax.numpy as jnp
from jax import lax
from jax.experimental import pallas as pl
from jax.experimental.pallas import tpu as pltpu
from jax.experimental.pallas import tpu_sc as plsc

F32 = jnp.float32
NPIX = 56 * 56
NTOK = 8 * NPIX
EMB = 64
NEMB = 512


def _dot(a, b):
    return jnp.dot(a, b, preferred_element_type=F32)


def _t2d(v):
    a, b = v.shape
    bp = -(-b // 128) * 128
    if bp != b:
        v = jnp.concatenate([v, jnp.zeros((a, bp - b), v.dtype)], axis=1)
    t = jnp.transpose(v)
    return t[:b] if bp != b else t


def _padf_cl(h):
    c = h.shape[1]
    z = jnp.zeros((57, c), F32)
    return jnp.concatenate([z, h, z], axis=0)


def _c3f_cl(hp, w, m0, m2):
    acc = None
    for dy in range(3):
        for dx in range(3):
            s = hp[57 + 56 * (dy - 1) + (dx - 1):][:NPIX, :]
            if dx == 0:
                s = s * m0
            elif dx == 2:
                s = s * m2
            t = _dot(s, w[dy, dx])
            acc = t if acc is None else acc + t
    return acc


def _res_cl(h, blocks, m0, m2):
    for wa, wb in blocks:
        t = jnp.maximum(h, 0.0)
        t = _c3f_cl(_padf_cl(t), wa, m0, m2)
        t = jnp.maximum(t, 0.0)
        t = _dot(t, wb)
        h = h + t
    return jnp.maximum(h, 0.0)


def _padf_cf(h):
    c = h.shape[0]
    z = jnp.zeros((c, 57), F32)
    return jnp.concatenate([z, h, z], axis=1)


def _c3f_cf(hp, w, m0, m2):
    acc = None
    for dy in range(3):
        for dx in range(3):
            s = hp[:, 57 + 56 * (dy - 1) + (dx - 1):][:, :NPIX]
            if dx == 0:
                s = s * m0
            elif dx == 2:
                s = s * m2
            t = _dot(w[dy, dx], s)
            acc = t if acc is None else acc + t
    return acc


def _res_cf(h, blocks, m0, m2):
    for wa, wb in blocks:
        t = jnp.maximum(h, 0.0)
        t = _c3f_cf(_padf_cf(t), wa, m0, m2)
        t = jnp.maximum(t, 0.0)
        t = _dot(wb, t)
        h = h + t
    return jnp.maximum(h, 0.0)


def _enc_body(xf_ref, w1, b1, w2, b2, w3, b3, r1a, r1b, r2a, r2b,
              wpv, bpv, z_ref, h1p_s):
    pats = []
    for dy in range(4):
        for dx in range(4):
            s0 = (dy // 2) * 113 + (dx // 2)
            pats.append(xf_ref[0, dy % 2, dx % 2, s0:s0 + 12656])
    p = jnp.stack(pats, axis=0)
    h1 = jnp.maximum(_dot(w1[...], p) + b1[...], 0.0)
    h1 = _t2d(h1).reshape(112, 113, 64)[:, :112, :]
    zr = jnp.zeros((1, 112, 64), F32)
    h1 = jnp.concatenate([zr, h1, zr], axis=0)
    zc = jnp.zeros((114, 1, 64), F32)
    h1p_s[:, :, 0:64] = jnp.concatenate([zc, h1, zc], axis=1)
    acc = None
    for dy in range(4):
        for dx in range(4):
            s = h1p_s[dy:dy + 111:2, dx:dx + 111:2, 0:64].reshape(NPIX, 64)
            t = _dot(s, w2[dy, dx])
            acc = t if acc is None else acc + t
    h = jnp.maximum(acc + b2[...], 0.0)
    r = lax.broadcasted_iota(jnp.int32, (NPIX, 1), 0) % 56
    m0 = (r > 0).astype(F32)
    m2 = (r < 55).astype(F32)
    h = _c3f_cl(_padf_cl(h), w3[...], m0, m2) + b3[...]
    h = _res_cl(h, ((r1a[...], r1b[...]), (r2a[...], r2b[...])), m0, m2)
    z = _dot(h, wpv[...]) + bpv[...]
    z_ref[0] = _t2d(z)


def _vq_body(z_ref, cb_ref, cbt_ref, idx_ref, q_ref, perp_ref, cnt_ref):
    i = pl.program_id(0)
    z2 = z_ref[0]
    cb = cb_ref[...]
    csq = jnp.sum(cb * cb, axis=1, keepdims=True)
    d = csq - 2.0 * jnp.dot(cb, z2, preferred_element_type=F32,
                            precision=lax.Precision.HIGHEST)
    dmin = jnp.min(d, axis=0, keepdims=True)
    i512 = lax.broadcasted_iota(jnp.int32, (NEMB, NPIX), 0)
    idx = jnp.min(jnp.where(d <= dmin, i512, NEMB), axis=0, keepdims=True)
    idx_ref[0] = idx
    onehot = (i512 == idx).astype(F32)
    q_ref[0] = _dot(cbt_ref[...], onehot)
    cnt = jnp.sum(onehot, axis=1, keepdims=True)

    @pl.when(i == 0)
    def _():
        cnt_ref[...] = cnt

    @pl.when(i > 0)
    def _():
        cnt_ref[...] = cnt_ref[...] + cnt

    @pl.when(i == pl.num_programs(0) - 1)
    def _():
        p = cnt_ref[...] / float(NTOK)
        perp_ref[...] = jnp.exp(-jnp.sum(p * jnp.log(p + 1e-10),
                                         keepdims=True).reshape(1, 1))


_T_TAPS = (((0, 1), (-1, 3)), ((1, 0), (0, 2)))
_CT2 = ((((1, 0, 0), (3, 1, -1)), ((0, 1, 0), (2, 0, 0))),
        (((1, 1, 0), (3, 0, 0)), ((0, 0, 1), (2, 1, 0))))


def _dec_body(q_ref, z_ref, wd, bd, r1a, r1b, r2a, r2b, wt1, bt1,
              w2a, bt2, xr_ref, loss_ref, sse_ref):
    i = pl.program_id(0)
    q = q_ref[0]
    zl = z_ref[0]
    dq = q - zl
    sse = jnp.sum(dq * dq)

    @pl.when(i == 0)
    def _():
        sse_ref[0] = sse

    @pl.when(i > 0)
    def _():
        sse_ref[0] = sse_ref[0] + sse

    c = lax.broadcasted_iota(jnp.int32, (1, NPIX), 1) % 56
    m0 = (c > 0).astype(F32)
    m2 = (c < 55).astype(F32)
    h = _c3f_cf(_padf_cf(q), wd[...], m0, m2) + bd[...]
    h = _res_cf(h, ((r1a[...], r1b[...]), (r2a[...], r2b[...])), m0, m2)
    hp = _padf_cf(h)
    par = {}
    for ry in range(2):
        for rx in range(2):
            acc = None
            for ey, dy in _T_TAPS[ry]:
                for ex, dx in _T_TAPS[rx]:
                    s = hp[:, 57 + 56 * ey + ex:][:, :NPIX]
                    if ex == -1:
                        s = s * m0
                    elif ex == 1:
                        s = s * m2
                    t = _dot(wt1[dy, dx], s)
                    acc = t if acc is None else acc + t
            par[(ry, rx)] = jnp.maximum(acc + bt1[...], 0.0)
    y = {k: _padf_cf(_dot(w2a[...], v)) for k, v in par.items()}
    for ty in range(2):
        for sy in range(2):
            for tx in range(2):
                for sx in range(2):
                    acc = None
                    for dy, py, ky in _CT2[ty][sy]:
                        for dx, px, kx in _CT2[tx][sx]:
                            yp = y[(py, px)]
                            k0 = 8 * (dy * 4 + dx)
                            s = yp[k0:k0 + 3, 57 + 56 * ky + kx:][:, :NPIX]
                            if kx == -1:
                                s = s * m0
                            elif kx == 1:
                                s = s * m2
                            acc = s if acc is None else acc + s
                    xr_ref[0, :, 2 * ty + sy, 2 * tx + sx] = acc + bt2[...]

    @pl.when(i == pl.num_programs(0) - 1)
    def _():
        loss_ref[...] = jnp.full((1, 1), 0.25 / float(NTOK * EMB),
                                 F32) * sse_ref[0]


def _full_spec(shape):
    nd = len(shape)
    return pl.BlockSpec(shape, lambda i, _n=nd: (0,) * _n)


def _batch_spec(shape):
    nd = len(shape)
    return pl.BlockSpec((1,) + shape,
                        lambda i, _n=nd: (i,) + (0,) * _n)


def kernel(x, e1_w, e1_b, e2_w, e2_b, e3_w, e3_b, er1_w1, er1_w2, er2_w1,
           er2_w2, pv_w, pv_b, codebook, d1_w, d1_b, dr1_w1, dr1_w2, dr2_w1,
           dr2_w2, dt1_w, dt1_b, dt2_w, dt2_b):
    f = F32

    def hwio(w):
        return jnp.transpose(w, (2, 3, 1, 0))

    def hwoi(w):
        return jnp.transpose(w, (2, 3, 0, 1))

    xpad = jnp.pad(x[:, 0], ((0, 0), (1, 1), (1, 1)))
    xpp = jnp.transpose(xpad.reshape(8, 113, 2, 113, 2),
                        (0, 2, 4, 1, 3)).reshape(8, 2, 2, 113 * 113)
    xpp = jnp.pad(xpp, ((0, 0), (0, 0), (0, 0), (0, 12800 - 113 * 113)))

    z = pl.pallas_call(
        _enc_body,
        grid=(8,),
        in_specs=[pl.BlockSpec((1, 2, 2, 12800),
                               lambda i: (i, 0, 0, 0)),
                  _full_spec((64, 16)), _full_spec((64, 1)),
                  _full_spec((4, 4, 64, 128)), _full_spec((1, 128)),
                  _full_spec((3, 3, 128, 128)), _full_spec((1, 128)),
                  _full_spec((3, 3, 128, 32)),
                  _full_spec((32, 128)),
                  _full_spec((3, 3, 128, 32)),
                  _full_spec((32, 128)),
                  _full_spec((128, 64)), _full_spec((1, 64))],
        out_specs=_batch_spec((EMB, NPIX)),
        out_shape=jax.ShapeDtypeStruct((8, EMB, NPIX), f),
        scratch_shapes=[pltpu.VMEM((114, 114, 128), f)],
    )(xpp, e1_w[:, 0].reshape(64, 16), e1_b.reshape(64, 1),
      hwio(e2_w), e2_b.reshape(1, 128),
      hwio(e3_w), e3_b.reshape(1, 128),
      hwio(er1_w1), er1_w2[:, :, 0, 0].T,
      hwio(er2_w1), er2_w2[:, :, 0, 0].T,
      pv_w[:, :, 0, 0].T, pv_b.reshape(1, 64))

    idx, q, perp = pl.pallas_call(
        _vq_body,
        grid=(8,),
        in_specs=[_batch_spec((EMB, NPIX)),
                  _full_spec((NEMB, EMB)),
                  _full_spec((EMB, NEMB))],
        out_specs=[pl.BlockSpec((1, 1, NPIX), lambda i: (i, 0, 0)),
                   _batch_spec((EMB, NPIX)),
                   _full_spec((1, 1))],
        out_shape=[jax.ShapeDtypeStruct((8, 1, NPIX), jnp.int32),
                   jax.ShapeDtypeStruct((8, EMB, NPIX), f),
                   jax.ShapeDtypeStruct((1, 1), f)],
        scratch_shapes=[pltpu.VMEM((NEMB, 1), f)],
    )(z, codebook, codebook.T)

    w2a = jnp.transpose(dt2_w, (2, 3, 1, 0))
    w2a = jnp.concatenate([w2a, jnp.zeros((4, 4, 5, 64), f)], axis=2)
    w2a = w2a.reshape(128, 64)

    xr, loss = pl.pallas_call(
        _dec_body,
        grid=(8,),
        in_specs=[_batch_spec((EMB, NPIX)),
                  _batch_spec((EMB, NPIX)),
                  _full_spec((3, 3, 128, 64)), _full_spec((128, 1)),
                  _full_spec((3, 3, 32, 128)),
                  _full_spec((128, 32)),
                  _full_spec((3, 3, 32, 128)),
                  _full_spec((128, 32)),
                  _full_spec((4, 4, 64, 128)), _full_spec((64, 1)),
                  _full_spec((128, 64)), _full_spec((3, 1))],
        out_specs=[_batch_spec((3, 4, 4, NPIX)), _full_spec((1, 1))],
        out_shape=[jax.ShapeDtypeStruct((8, 3, 4, 4, NPIX), f),
                   jax.ShapeDtypeStruct((1, 1), f)],
        scratch_shapes=[pltpu.SMEM((1,), f)],
    )(q, z, hwoi(d1_w), d1_b.reshape(128, 1),
      hwoi(dr1_w1), dr1_w2[:, :, 0, 0],
      hwoi(dr2_w1), dr2_w2[:, :, 0, 0],
      jnp.transpose(dt1_w, (2, 3, 1, 0)), dt1_b.reshape(64, 1),
      w2a, dt2_b.reshape(3, 1))

    r = xr.reshape(8, 3, 4, 4, 56, 56)
    x_recon = jnp.transpose(r, (0, 1, 4, 2, 5, 3)).reshape(8, 3, 224, 224)

    return (loss[0, 0], x_recon, perp[0, 0])
